# baseline (speedup 1.0000x reference)
import jax
import jax.numpy as jnp
from jax import lax
from jax.experimental import pallas as pl
from jax.experimental.pallas import tpu as pltpu
from jax.experimental.pallas import tpu_sc as plsc

B, N, S, D = 8, 14, 196, 128
NW = 32
R = 6
RLO = NW * R
RB = S - RLO
NGRP = 49
NJ = 6 * NGRP
BAND_K = 10


def _body(x_hbm, out_hbm, buf, band, sem_in, sem_lt, sem_bd, sem_out):
    c = lax.axis_index("c")
    s = lax.axis_index("s")
    wid = s * 2 + c
    lo = wid * R

    jlo = jnp.minimum((wid * NJ) // NW, NJ - BAND_K)
    jhi = jlo + BAND_K
    g0 = jlo // 6
    ghi = (jhi - 1) // 6

    for n in range(8, N):
        pltpu.async_copy(x_hbm.at[n, pl.ds(lo, R)], buf.at[n], sem_in)
    for n in range(8):
        pltpu.async_copy(x_hbm.at[n, pl.ds(lo, R)], buf.at[n], sem_lt)
    for p in range(6):
        pltpu.async_copy(x_hbm.at[8 + p, pl.ds(RLO, RB)], band.at[p], sem_bd)
    for t in range(3):
        gt = g0 + t

        @pl.when((gt >= 1) & (gt <= ghi))
        def _():
            pltpu.async_copy(
                x_hbm.at[lax.div(gt - 1, 6), pl.ds(RLO, RB)],
                band.at[6 + t],
                sem_bd,
            )

    pltpu.make_async_copy(
        x_hbm.at[pl.ds(8, 6), pl.ds(lo, R)], buf.at[pl.ds(8, 6)], sem_in
    ).wait()

    def fire(g, carry):
        for p in range(6):
            keep = (g == 0) | (lax.rem(g - 1, 6) != p)

            @pl.when(keep)
            def _():
                j = g * 6 + p
                pltpu.async_copy(
                    buf.at[8 + p],
                    out_hbm.at[j // NGRP, lax.rem(j, NGRP), pl.ds(lo, R)],
                    sem_out,
                )

        return carry

    lax.fori_loop(0, NGRP, fire, 0)

    for p in range(6):
        pltpu.make_async_copy(
            x_hbm.at[8 + p, pl.ds(RLO, RB)], band.at[p], sem_bd
        ).wait()
    for t in range(3):
        gt = g0 + t

        @pl.when((gt >= 1) & (gt <= ghi))
        def _():
            pltpu.make_async_copy(
                x_hbm.at[0, pl.ds(RLO, RB)], band.at[6 + t], sem_bd
            ).wait()

    def fire_band(t, carry):
        j = jlo + t
        g = lax.div(j, 6)
        p = lax.rem(j, 6)
        replaced = (g > 0) & (lax.rem(g - 1, 6) == p)
        slot = jnp.where(replaced, 6 + (g - g0), p)
        pltpu.async_copy(
            band.at[slot],
            out_hbm.at[lax.div(j, NGRP), lax.rem(j, NGRP), pl.ds(RLO, RB)],
            sem_out,
        )
        return carry

    lax.fori_loop(0, BAND_K, fire_band, 0)

    pltpu.make_async_copy(
        x_hbm.at[pl.ds(0, 8), pl.ds(lo, R)], buf.at[pl.ds(0, 8)], sem_lt
    ).wait()

    def fire_light(g, carry):
        jj = lax.rem(g - 1, 6)
        j = g * 6 + jj
        pltpu.async_copy(
            buf.at[lax.div(g - 1, 6)],
            out_hbm.at[j // NGRP, lax.rem(j, NGRP), pl.ds(lo, R)],
            sem_out,
        )
        return carry

    lax.fori_loop(1, NGRP, fire_light, 0)

    for _ in range(6):
        pltpu.make_async_copy(
            out_hbm.at[0, pl.ds(0, NGRP), pl.ds(lo, R)],
            out_hbm.at[0, pl.ds(0, NGRP), pl.ds(lo, R)],
            sem_out,
        ).wait()

    pltpu.make_async_copy(
        out_hbm.at[0, pl.ds(0, BAND_K), pl.ds(RLO, RB)],
        out_hbm.at[0, pl.ds(0, BAND_K), pl.ds(RLO, RB)],
        sem_out,
    ).wait()


@jax.jit
def kernel(x):
    x_t = x.transpose(1, 2, 0, 3)
    out_t = pl.kernel(
        _body,
        out_type=jax.ShapeDtypeStruct((6, NGRP, S, B, D), jnp.float32),
        mesh=plsc.VectorSubcoreMesh(core_axis_name="c", subcore_axis_name="s"),
        scratch_types=[
            pltpu.VMEM((N, R, B, D), jnp.float32),
            pltpu.VMEM((9, RB, B, D), jnp.float32),
            pltpu.SemaphoreType.DMA,
            pltpu.SemaphoreType.DMA,
            pltpu.SemaphoreType.DMA,
            pltpu.SemaphoreType.DMA,
        ],
    )(x_t)
    return out_t.transpose(3, 0, 1, 2, 4)
